# Initial kernel scaffold; baseline (speedup 1.0000x reference)
import jax
import jax.numpy as jnp
from jax.experimental import pallas as pl


def kernel(node_table, rel_table, We1, Wr1, a1, Wres1, We2, a2, Wres2, ent_ids, rel_ids, edge_index, batch_node_ids):
    raise NotImplementedError("write your pallas kernel here")


# speedup vs baseline: 1.0265x; 1.0265x over previous
import jax
import jax.numpy as jnp
from jax.experimental import pallas as pl

N = 10000
E = 320000
REL = 16
D_IN = 128
D_HID = 256
H = 8
DH = D_HID // H
HOP = 4
ALPHA = 0.15


def _elu_blk(x_ref, o_ref):
    x = x_ref[...]
    o_ref[...] = jnp.where(x > 0, x, jnp.exp(x) - 1.0)


def _elu_pallas(x):
    return pl.pallas_call(
        _elu_blk,
        out_shape=jax.ShapeDtypeStruct(x.shape, x.dtype),
        grid=(10,),
        in_specs=[pl.BlockSpec((x.shape[0] // 10, x.shape[1]), lambda i: (i, 0))],
        out_specs=pl.BlockSpec((x.shape[0] // 10, x.shape[1]), lambda i: (i, 0)),
    )(x)


def _layer(h, src, dst, We, a, Wres, rel_ids=None, rel_table=None, Wr=None):
    feat = (h @ We).reshape(N, H, DH)
    u = (feat * a[0]).sum(-1)
    v = (feat * a[1]).sum(-1)
    if rel_ids is not None:
        w = ((rel_table @ Wr).reshape(-1, H, DH) * a[2]).sum(-1)
        p = u[src] + v[dst] + w[rel_ids]
    else:
        p = u[src] + v[dst]
    ex = jnp.exp(jnp.where(p > 0, p, 0.2 * p))
    s = jax.ops.segment_sum(ex, dst, num_segments=N)
    r = (1.0 - ALPHA) / (s + 1e-16)
    f = feat
    for _ in range(HOP):
        g = jax.ops.segment_sum(ex[:, :, None] * f[src], dst, num_segments=N)
        f = ALPHA * feat + r[:, :, None] * g
    return _elu_pallas(f.reshape(N, H * DH) + h @ Wres)


def kernel(node_table, rel_table, We1, Wr1, a1, Wres1, We2, a2, Wres2,
           ent_ids, rel_ids, edge_index, batch_node_ids):
    h0 = node_table[ent_ids]
    src, dst = edge_index[0], edge_index[1]
    h1 = _layer(h0, src, dst, We1, a1, Wres1, rel_ids, rel_table, Wr1)
    h2 = _layer(h1, src, dst, We2, a2, Wres2)
    return h2[batch_node_ids]


# speedup vs baseline: 25.9442x; 25.2750x over previous
import functools
import jax
import jax.numpy as jnp
from jax import lax
from jax.experimental import pallas as pl
from jax.experimental.pallas import tpu as pltpu, tpu_sc as plsc

N = 10000
NP = 10240
E = 320000
EP = 327680
EW = EP // 16
REL = 16
H = 8
HH = 4
DH = 32
HOP = 4
ALPHA = 0.15
NC, NS = 2, 16
CS = 1024
CH = 256
NW = NP // NS


def _tc_proj_body(h_ref, we_ref, wres_ref, a0_ref, a1_ref,
                  f0_ref, f1_ref, u_ref, v_ref, res_ref):
    h = h_ref[...]
    feat = jnp.dot(h, we_ref[...], preferred_element_type=jnp.float32)
    f0_ref[...] = feat[:, :64]
    f1_ref[...] = feat[:, 64:]
    u_ref[...] = jnp.dot(feat, a0_ref[0], preferred_element_type=jnp.float32)
    v_ref[...] = jnp.dot(feat, a1_ref[0], preferred_element_type=jnp.float32)
    res_ref[...] = jnp.dot(h, wres_ref[...], preferred_element_type=jnp.float32)


def _tc_proj(h, We, Wres, A0, A1):
    din = h.shape[1]
    blk = 1280
    nb = NP // blk
    f0, f1, u, v, res = pl.pallas_call(
        _tc_proj_body,
        grid=(nb, 2),
        in_specs=[
            pl.BlockSpec((blk, din), lambda i, j: (i, 0)),
            pl.BlockSpec((din, 128), lambda i, j: (0, j)),
            pl.BlockSpec((din, 128), lambda i, j: (0, j)),
            pl.BlockSpec((1, 128, 16), lambda i, j: (j, 0, 0)),
            pl.BlockSpec((1, 128, 16), lambda i, j: (j, 0, 0)),
        ],
        out_specs=[
            pl.BlockSpec((blk, 64), lambda i, j: (j * nb + i, 0)),
            pl.BlockSpec((blk, 64), lambda i, j: (j * nb + i, 0)),
            pl.BlockSpec((blk, 16), lambda i, j: (j * nb + i, 0)),
            pl.BlockSpec((blk, 16), lambda i, j: (j * nb + i, 0)),
            pl.BlockSpec((blk, 128), lambda i, j: (i, j)),
        ],
        out_shape=[
            jax.ShapeDtypeStruct((2 * NP, 64), jnp.float32),
            jax.ShapeDtypeStruct((2 * NP, 64), jnp.float32),
            jax.ShapeDtypeStruct((2 * NP, 16), jnp.float32),
            jax.ShapeDtypeStruct((2 * NP, 16), jnp.float32),
            jax.ShapeDtypeStruct((NP, 256), jnp.float32),
        ],
    )(h, We, Wres, A0, A1)
    return (f0, f1), u, v, res


def _tc_rel_body(rt_ref, wr_ref, a2_ref, o_ref):
    rfeat = jnp.dot(rt_ref[...], wr_ref[...], preferred_element_type=jnp.float32)
    o_ref[...] = jnp.dot(rfeat, a2_ref[...], preferred_element_type=jnp.float32)


def _tc_rel(rel_table, Wr, A2):
    return pl.pallas_call(
        _tc_rel_body,
        out_shape=jax.ShapeDtypeStruct((REL, 16), jnp.float32),
    )(rel_table, Wr, A2)


def _tc_rb_body(s_ref, k2_ref, o0_ref, o1_ref):
    s = s_ref[...][:, :4]
    r = (1.0 - ALPHA) / (s + 1e-16)
    k2 = k2_ref[...]
    o0_ref[...] = jnp.dot(r[:, :2], k2, preferred_element_type=jnp.float32)
    o1_ref[...] = jnp.dot(r[:, 2:], k2, preferred_element_type=jnp.float32)


def _tc_rb(S, K2):
    blk = 1280
    return pl.pallas_call(
        _tc_rb_body,
        grid=(2 * NP // blk,),
        in_specs=[pl.BlockSpec((blk, 16), lambda i: (i, 0)),
                  pl.BlockSpec((2, 64), lambda i: (0, 0))],
        out_specs=[pl.BlockSpec((blk, 64), lambda i: (i, 0)),
                   pl.BlockSpec((blk, 64), lambda i: (i, 0))],
        out_shape=[jax.ShapeDtypeStruct((2 * NP, 64), jnp.float32),
                   jax.ShapeDtypeStruct((2 * NP, 64), jnp.float32)],
    )(S, K2)


def _elu(x):
    return jnp.where(x > 0, x, jnp.exp(x) - 1.0)


def _tc_bridge_body(f00_ref, f10_ref, f01_ref, f11_ref, res_ref, o_ref):
    f = jnp.concatenate([f00_ref[...], f10_ref[...],
                         f01_ref[...], f11_ref[...]], axis=1)
    o_ref[...] = _elu(f + res_ref[...])


def _tc_bridge(FBp, RES):
    blk = 1280
    nb = NP // blk
    return pl.pallas_call(
        _tc_bridge_body,
        grid=(nb,),
        in_specs=[
            pl.BlockSpec((blk, 64), lambda i: (i, 0)),
            pl.BlockSpec((blk, 64), lambda i: (i, 0)),
            pl.BlockSpec((blk, 64), lambda i: (nb + i, 0)),
            pl.BlockSpec((blk, 64), lambda i: (nb + i, 0)),
            pl.BlockSpec((blk, 256), lambda i: (i, 0)),
        ],
        out_specs=pl.BlockSpec((blk, 256), lambda i: (i, 0)),
        out_shape=jax.ShapeDtypeStruct((NP, 256), jnp.float32),
    )(FBp[0], FBp[1], FBp[0], FBp[1], RES)


def _tc_final_body(f00_ref, f10_ref, f01_ref, f11_ref, res_ref, ids_ref, o_ref):
    i = pl.program_id(0)
    f = jnp.concatenate([f00_ref[...], f10_ref[...],
                         f01_ref[...], f11_ref[...]], axis=1)
    h2 = _elu(f + res_ref[...])
    ids = ids_ref[...]
    col = lax.broadcasted_iota(jnp.int32, (64, 1280), 1) + i * 1280
    onehot = (ids == col).astype(jnp.float32)
    part = jnp.dot(onehot, h2, preferred_element_type=jnp.float32)

    @pl.when(i == 0)
    def _():
        o_ref[...] = part

    @pl.when(i > 0)
    def _():
        o_ref[...] += part


def _tc_final(FBp, RES, ids2d):
    blk = 1280
    nb = NP // blk
    return pl.pallas_call(
        _tc_final_body,
        grid=(nb,),
        in_specs=[
            pl.BlockSpec((blk, 64), lambda i: (i, 0)),
            pl.BlockSpec((blk, 64), lambda i: (i, 0)),
            pl.BlockSpec((blk, 64), lambda i: (nb + i, 0)),
            pl.BlockSpec((blk, 64), lambda i: (nb + i, 0)),
            pl.BlockSpec((blk, 256), lambda i: (i, 0)),
            pl.BlockSpec((64, 1), lambda i: (0, 0)),
        ],
        out_specs=pl.BlockSpec((64, 256), lambda i: (0, 0)),
        out_shape=jax.ShapeDtypeStruct((64, 256), jnp.float32),
    )(FBp[0], FBp[1], FBp[0], FBp[1], RES, ids2d)


_MESH = plsc.VectorSubcoreMesh(core_axis_name="c", subcore_axis_name="s")


@functools.partial(
    pl.kernel,
    out_type=jax.ShapeDtypeStruct((NP, 128), jnp.float32),
    mesh=_MESH,
    scratch_types=[
        pltpu.VMEM((5, 64), jnp.int32),
        pltpu.VMEM((320, 128), jnp.float32),
        pltpu.SemaphoreType.DMA,
    ],
)
def _sc_embed(table_hbm, idx_hbm, out_hbm, idx_v, rows_v, sem):
    w = lax.axis_index("s") * NC + lax.axis_index("c")
    pltpu.sync_copy(idx_hbm.at[w], idx_v)
    for j in range(5):
        pltpu.async_copy(table_hbm.at[idx_v.at[j]],
                         rows_v.at[pl.ds(64 * j, 64)], sem)
    for _ in range(5):
        pltpu.make_async_copy(table_hbm.at[idx_v.at[0]],
                              rows_v.at[pl.ds(0, 64)], sem).wait()
    pltpu.sync_copy(rows_v, out_hbm.at[pl.ds(w * 320, 320)])


_LANE4 = None


def _make_sc_score(userel):
    @functools.partial(
        pl.kernel,
        out_type=(jax.ShapeDtypeStruct((2, EP, 16), jnp.float32),
                  jax.ShapeDtypeStruct((2 * NP, 16), jnp.float32)),
        mesh=_MESH,
        scratch_types=[
            pltpu.VMEM((CS,), jnp.int32),
            pltpu.VMEM((CS,), jnp.int32),
            pltpu.VMEM((CS,), jnp.int32),
            pltpu.VMEM((CS // 128, 128), jnp.int32),
            pltpu.VMEM((CS, 16), jnp.float32),
            pltpu.VMEM((CS, 16), jnp.float32),
            pltpu.VMEM((CS, 16), jnp.float32),
            pltpu.VMEM((CS, 16), jnp.float32),
            pltpu.SemaphoreType.DMA,
            pltpu.VMEM_SHARED((NP, 16), jnp.float32),
        ],
        compiler_params=pltpu.CompilerParams(use_tc_tiling_on_sc=False),
    )
    def sc_score(U, V, W16, SRCG, DSTG, DSTP2, RELG, Z16,
                 EX, S, srcb, dstb, relb, dstb2, urows, vrows, wrows,
                 exrows, sem, s_sh):
        c = lax.axis_index("c")
        sid = lax.axis_index("s")
        iota = lax.iota(jnp.int32, 16)
        lane4 = iota < 4
        nrow0 = pl.multiple_of(sid * NW, 128)
        crow0 = pl.multiple_of(c * NP, 128)

        pltpu.sync_copy(Z16.at[pl.ds(nrow0, NW)], s_sh.at[pl.ds(nrow0, NW)])
        plsc.subcore_barrier()

        def chunk(k, _):
            b = pl.multiple_of(sid * EW + k * CS, CS)
            pltpu.sync_copy(SRCG.at[c, pl.ds(b, CS)], srcb)
            pltpu.sync_copy(DSTG.at[c, pl.ds(b, CS)], dstb)
            if userel:
                pltpu.sync_copy(RELG.at[c, pl.ds(b, CS)], relb)
            pltpu.sync_copy(
                DSTP2.at[pl.ds(pl.multiple_of(b // 128, 8), CS // 128)], dstb2)

            ng = CS // 128
            for j in range(ng):
                sl = pl.ds(128 * j, 128)
                pltpu.async_copy(U.at[srcb.at[sl]], urows.at[sl], sem)
                pltpu.async_copy(V.at[dstb.at[sl]], vrows.at[sl], sem)
                if userel:
                    pltpu.async_copy(W16.at[relb.at[sl]], wrows.at[sl], sem)
            for _j in range(ng * (3 if userel else 2)):
                pltpu.make_async_copy(U.at[srcb.at[pl.ds(0, 128)]],
                                      urows.at[pl.ds(0, 128)], sem).wait()

            def edge(j, _):
                p = urows[j, pl.ds(0, 16)] + vrows[j, pl.ds(0, 16)]
                if userel:
                    p = p + wrows[j, pl.ds(0, 16)]
                p = jnp.where(p > 0, p, 0.2 * p)
                e = jnp.exp(p)
                e = jnp.where(lane4, e, 0.0)
                e = jnp.where(b + j < E, e, jnp.zeros((16,), jnp.float32))
                exrows[j, pl.ds(0, 16)] = e
                return _
            lax.fori_loop(0, CS, edge, None)

            pltpu.sync_copy(exrows, EX.at[c, pl.ds(b, CS)])
            for j in range(ng):
                pltpu.sync_copy(exrows.at[pl.ds(128 * j, 128)],
                                s_sh.at[dstb2.at[j]], add=True)
            return _
        lax.fori_loop(0, EW // CS, chunk, None)

        plsc.subcore_barrier()
        pltpu.sync_copy(s_sh.at[pl.ds(nrow0, NW)],
                        S.at[pl.ds(pl.multiple_of(c * NP + sid * NW, 128), NW)])

    return sc_score


_sc_score_rel = _make_sc_score(True)
_sc_score_norel = _make_sc_score(False)


@functools.partial(
    pl.kernel,
    out_type=tuple(jax.ShapeDtypeStruct((2 * NP, 64), jnp.float32)
                   for _ in range(4)),
    mesh=_MESH,
    scratch_types=[
        pltpu.VMEM((2, 128), jnp.int32),
        pltpu.VMEM((2, 128), jnp.int32),
        pltpu.VMEM((CH, 16), jnp.float32),
        pltpu.VMEM((CH, 64), jnp.float32),
        pltpu.VMEM((128, 64), jnp.float32),
        pltpu.VMEM((128, 64), jnp.float32),
        pltpu.VMEM((128, 64), jnp.float32),
        pltpu.SemaphoreType.DMA,
        pltpu.VMEM_SHARED((NP, 64), jnp.float32),
    ],
    compiler_params=pltpu.CompilerParams(use_tc_tiling_on_sc=False),
)
def _sc_hops(F0, F1, RB0, RB1, EX, SRCG3, DSTP2, ZG, fA0, fA1, fB0, fB1,
             srcb2, dstb2, exb, rows, gb, featb, rbb, sem, g_sh):
    c = lax.axis_index("c")
    sid = lax.axis_index("s")
    nrow0 = pl.multiple_of(sid * NW, 128)
    crow0 = pl.multiple_of(c * NP, 128)

    pltpu.sync_copy(ZG.at[pl.ds(nrow0, NW)], g_sh.at[pl.ds(nrow0, NW)])
    plsc.subcore_barrier()

    FS = ((F0, F1), (fA0, fA1), (fB0, fB1), (fA0, fA1))
    FD = ((fA0, fA1), (fB0, fB1), (fA0, fA1), (fB0, fB1))
    RBp = (RB0, RB1)
    FEATp = (F0, F1)

    for hop in range(HOP):
        for p in range(2):
            fsrc = FS[hop][p]
            fdst = FD[hop][p]

            def chunk(k, _, fsrc=fsrc, p=p):
                b = pl.multiple_of(sid * EW + k * CH, CH)
                r0 = pl.multiple_of(b // 128, 2)
                pltpu.sync_copy(SRCG3.at[c, pl.ds(r0, 2)], srcb2)
                pltpu.sync_copy(DSTP2.at[pl.ds(r0, 2)], dstb2)
                pltpu.sync_copy(EX.at[c, pl.ds(b, CH)], exb)
                for j in range(2):
                    pltpu.async_copy(fsrc.at[srcb2.at[j]],
                                     rows.at[pl.ds(128 * j, 128)], sem)
                for _j in range(2):
                    pltpu.make_async_copy(fsrc.at[srcb2.at[0]],
                                          rows.at[pl.ds(0, 128)], sem).wait()

                def edge(j, _):
                    exv = exb[j, pl.ds(0, 16)]
                    for t2 in range(2):
                        sc = exv[2 * p + t2]
                        for q in range(2):
                            sl = pl.ds(t2 * 32 + q * 16, 16)
                            rows[j, sl] = rows[j, sl] * sc
                    return _
                lax.fori_loop(0, CH, edge, None)

                for j in range(2):
                    pltpu.sync_copy(rows.at[pl.ds(128 * j, 128)],
                                    g_sh.at[dstb2.at[j]], add=True)
                return _
            lax.fori_loop(0, EW // CH, chunk, None)
            plsc.subcore_barrier()

            for q in range(NW // 128):
                r0n = pl.multiple_of(nrow0 + q * 128, 128)
                fr = pl.multiple_of(crow0 + nrow0 + q * 128, 128)
                pltpu.sync_copy(g_sh.at[pl.ds(r0n, 128)], gb)
                pltpu.sync_copy(FEATp[p].at[pl.ds(fr, 128)], featb)
                pltpu.sync_copy(RBp[p].at[pl.ds(fr, 128)], rbb)

                def row(i, _):
                    for m in range(4):
                        sl = pl.ds(m * 16, 16)
                        gb[i, sl] = (ALPHA * featb[i, sl]
                                     + rbb[i, sl] * gb[i, sl])
                    return _
                lax.fori_loop(0, 128, row, None)
                pltpu.sync_copy(gb, fdst.at[pl.ds(fr, 128)])
                if not (hop == HOP - 1 and p == 1):
                    pltpu.sync_copy(ZG.at[pl.ds(r0n, 128)],
                                    g_sh.at[pl.ds(r0n, 128)])
            plsc.subcore_barrier()


def _prep_weights(a):
    eye = jnp.eye(H, dtype=jnp.float32)
    def bd(ah):
        full = (eye[:, None, :] * ah[:, :, None]).reshape(H * DH, H)
        halves = jnp.stack([full[:128, :4], full[128:, 4:]])
        return jnp.pad(halves, ((0, 0), (0, 0), (0, 12)))
    return bd(a[0]), bd(a[1])


def kernel(node_table, rel_table, We1, Wr1, a1, Wres1, We2, a2, Wres2,
           ent_ids, rel_ids, edge_index, batch_node_ids):
    src = edge_index[0].astype(jnp.int32)
    dst = edge_index[1].astype(jnp.int32)
    rel = rel_ids.astype(jnp.int32)

    enti = jnp.pad(ent_ids.astype(jnp.int32), (0, NP - N)).reshape(32, 5, 64)
    A0_1, A1_1 = _prep_weights(a1)
    A2_1 = jnp.pad((jnp.eye(H, dtype=jnp.float32)[:, None, :]
                    * a1[2][:, :, None]).reshape(H * DH, H), ((0, 0), (0, 8)))
    A0_2, A1_2 = _prep_weights(a2)
    K2 = jnp.kron(jnp.eye(2, dtype=jnp.float32), jnp.ones((1, 32), jnp.float32))
    ids2d = batch_node_ids.astype(jnp.int32).reshape(64, 1)

    srcp = jnp.pad(src, (0, EP - E))
    dstp = jnp.pad(dst, (0, EP - E))
    relp = jnp.pad(rel, (0, EP - E))
    srcg = jnp.stack([srcp, srcp + NP])
    dstg = jnp.stack([dstp, dstp + NP])
    relg = jnp.stack([relp, relp])
    srcg3 = srcg.reshape(2, EP // 128, 128)
    dstp2 = dstp.reshape(EP // 128, 128)
    z16 = jnp.zeros((NP, 16), jnp.float32)
    zg = jnp.zeros((NP, 64), jnp.float32)
    w16_zero = jnp.zeros((REL, 16), jnp.float32)

    ENT = _sc_embed(node_table, enti)

    w16_1 = _tc_rel(rel_table, Wr1, A2_1)
    w16c_1 = jnp.concatenate(
        [jnp.pad(w16_1[:, :4], ((0, 0), (0, 12))),
         jnp.pad(w16_1[:, 4:8], ((0, 0), (0, 12)))], axis=0)
    FEATp, U, V, RES1 = _tc_proj(ENT, We1, Wres1, A0_1, A1_1)
    relg1 = jnp.stack([relp, relp + REL])
    EX1, S1 = _sc_score_rel(U, V, w16c_1, srcg, dstg, dstp2, relg1, z16)
    RB0, RB1 = _tc_rb(S1, K2)
    _, _, FBa, FBb = _sc_hops(FEATp[0], FEATp[1], RB0, RB1, EX1,
                              srcg3, dstp2, zg)
    H1 = _tc_bridge((FBa, FBb), RES1)

    FEATq, Ub, Vb, RES2 = _tc_proj(H1, We2, Wres2, A0_2, A1_2)
    EX2, S2 = _sc_score_norel(Ub, Vb, w16_zero, srcg, dstg, dstp2, relg, z16)
    RB0b, RB1b = _tc_rb(S2, K2)
    _, _, FBc, FBd = _sc_hops(FEATq[0], FEATq[1], RB0b, RB1b, EX2,
                              srcg3, dstp2, zg)

    return _tc_final((FBc, FBd), RES2, ids2d)


# speedup vs baseline: 34.9289x; 1.3463x over previous
import functools
import jax
import jax.numpy as jnp
from jax import lax
from jax.experimental import pallas as pl
from jax.experimental.pallas import tpu as pltpu, tpu_sc as plsc

N = 10000
NP = 10240
E = 320000
EP = 327680
EW = EP // 16
REL = 16
H = 8
HH = 4
DH = 32
HOP = 4
ALPHA = 0.15
NC, NS = 2, 16
CS = 1024
CH = 512
NW = NP // NS


def _tc_proj_body(h_ref, we_ref, wres_ref, a0_ref, a1_ref,
                  f0_ref, f1_ref, f2_ref, f3_ref, u_ref, v_ref, res_ref):
    h = h_ref[...]
    feat = jnp.dot(h, we_ref[...], preferred_element_type=jnp.float32)
    f0_ref[...] = feat[:, 0:32]
    f1_ref[...] = feat[:, 32:64]
    f2_ref[...] = feat[:, 64:96]
    f3_ref[...] = feat[:, 96:128]
    u_ref[...] = jnp.dot(feat, a0_ref[0], preferred_element_type=jnp.float32)
    v_ref[...] = jnp.dot(feat, a1_ref[0], preferred_element_type=jnp.float32)
    res_ref[...] = jnp.dot(h, wres_ref[...], preferred_element_type=jnp.float32)


def _tc_proj(h, We, Wres, A0, A1):
    din = h.shape[1]
    blk = 1280
    nb = NP // blk
    f0, f1, f2, f3, u, v, res = pl.pallas_call(
        _tc_proj_body,
        grid=(nb, 2),
        in_specs=[
            pl.BlockSpec((blk, din), lambda i, j: (i, 0)),
            pl.BlockSpec((din, 128), lambda i, j: (0, j)),
            pl.BlockSpec((din, 128), lambda i, j: (0, j)),
            pl.BlockSpec((1, 128, 16), lambda i, j: (j, 0, 0)),
            pl.BlockSpec((1, 128, 16), lambda i, j: (j, 0, 0)),
        ],
        out_specs=(
            [pl.BlockSpec((blk, 32), lambda i, j: (j * nb + i, 0))
             for _ in range(4)]
            + [pl.BlockSpec((blk, 16), lambda i, j: (j * nb + i, 0)),
               pl.BlockSpec((blk, 16), lambda i, j: (j * nb + i, 0)),
               pl.BlockSpec((blk, 128), lambda i, j: (i, j))]),
        out_shape=(
            [jax.ShapeDtypeStruct((2 * NP, 32), jnp.float32)
             for _ in range(4)]
            + [jax.ShapeDtypeStruct((2 * NP, 16), jnp.float32),
               jax.ShapeDtypeStruct((2 * NP, 16), jnp.float32),
               jax.ShapeDtypeStruct((NP, 256), jnp.float32)]),
    )(h, We, Wres, A0, A1)
    return (f0, f1, f2, f3), u, v, res


def _tc_rel_body(rt_ref, wr_ref, a2_ref, o_ref):
    rfeat = jnp.dot(rt_ref[...], wr_ref[...], preferred_element_type=jnp.float32)
    o_ref[...] = jnp.dot(rfeat, a2_ref[...], preferred_element_type=jnp.float32)


def _tc_rel(rel_table, Wr, A2):
    return pl.pallas_call(
        _tc_rel_body,
        out_shape=jax.ShapeDtypeStruct((REL, 16), jnp.float32),
    )(rel_table, Wr, A2)


def _tc_rb_body(s_ref, k1_ref, o0_ref, o1_ref, o2_ref, o3_ref):
    s = s_ref[...][:, :4]
    r = (1.0 - ALPHA) / (s + 1e-16)
    k1 = k1_ref[...]
    outs = (o0_ref, o1_ref, o2_ref, o3_ref)
    for p in range(4):
        outs[p][...] = jnp.dot(r[:, p:p + 1], k1,
                               preferred_element_type=jnp.float32)


def _tc_rb(S, K1):
    blk = 1280
    return pl.pallas_call(
        _tc_rb_body,
        grid=(2 * NP // blk,),
        in_specs=[pl.BlockSpec((blk, 16), lambda i: (i, 0)),
                  pl.BlockSpec((1, 32), lambda i: (0, 0))],
        out_specs=[pl.BlockSpec((blk, 32), lambda i: (i, 0))
                   for _ in range(4)],
        out_shape=[jax.ShapeDtypeStruct((2 * NP, 32), jnp.float32)
                   for _ in range(4)],
    )(S, K1)


def _elu(x):
    return jnp.where(x > 0, x, jnp.exp(x) - 1.0)


def _tc_bridge_body(*refs):
    fr = refs[:8]
    res_ref, o_ref = refs[8], refs[9]
    f = jnp.concatenate([r[...] for r in fr], axis=1)
    o_ref[...] = _elu(f + res_ref[...])


def _bridge_specs(blk, nb):
    lo = [pl.BlockSpec((blk, 32), lambda i: (i, 0)) for _ in range(4)]
    hi = [pl.BlockSpec((blk, 32), lambda i: (nb + i, 0)) for _ in range(4)]
    return lo + hi


def _tc_bridge(FBq, RES):
    blk = 1280
    nb = NP // blk
    return pl.pallas_call(
        _tc_bridge_body,
        grid=(nb,),
        in_specs=_bridge_specs(blk, nb) + [
            pl.BlockSpec((blk, 256), lambda i: (i, 0)),
        ],
        out_specs=pl.BlockSpec((blk, 256), lambda i: (i, 0)),
        out_shape=jax.ShapeDtypeStruct((NP, 256), jnp.float32),
    )(*FBq, *FBq, RES)


def _tc_final_body(*refs):
    fr = refs[:8]
    res_ref, ids_ref, o_ref = refs[8], refs[9], refs[10]
    i = pl.program_id(0)
    f = jnp.concatenate([r[...] for r in fr], axis=1)
    h2 = _elu(f + res_ref[...])
    ids = ids_ref[...]
    col = lax.broadcasted_iota(jnp.int32, (64, 1280), 1) + i * 1280
    onehot = (ids == col).astype(jnp.float32)
    part = jnp.dot(onehot, h2, preferred_element_type=jnp.float32)

    @pl.when(i == 0)
    def _():
        o_ref[...] = part

    @pl.when(i > 0)
    def _():
        o_ref[...] += part


def _tc_final(FBq, RES, ids2d):
    blk = 1280
    nb = NP // blk
    return pl.pallas_call(
        _tc_final_body,
        grid=(nb,),
        in_specs=_bridge_specs(blk, nb) + [
            pl.BlockSpec((blk, 256), lambda i: (i, 0)),
            pl.BlockSpec((64, 1), lambda i: (0, 0)),
        ],
        out_specs=pl.BlockSpec((64, 256), lambda i: (0, 0)),
        out_shape=jax.ShapeDtypeStruct((64, 256), jnp.float32),
    )(*FBq, *FBq, RES, ids2d)


_MESH = plsc.VectorSubcoreMesh(core_axis_name="c", subcore_axis_name="s")


@functools.partial(
    pl.kernel,
    out_type=jax.ShapeDtypeStruct((NP, 128), jnp.float32),
    mesh=_MESH,
    scratch_types=[
        pltpu.VMEM((5, 64), jnp.int32),
        pltpu.VMEM((320, 128), jnp.float32),
        pltpu.SemaphoreType.DMA,
    ],
)
def _sc_embed(table_hbm, idx_hbm, out_hbm, idx_v, rows_v, sem):
    w = lax.axis_index("s") * NC + lax.axis_index("c")
    pltpu.sync_copy(idx_hbm.at[w], idx_v)
    for j in range(5):
        pltpu.async_copy(table_hbm.at[idx_v.at[j]],
                         rows_v.at[pl.ds(64 * j, 64)], sem)
    for _ in range(5):
        pltpu.make_async_copy(table_hbm.at[idx_v.at[0]],
                              rows_v.at[pl.ds(0, 64)], sem).wait()
    pltpu.sync_copy(rows_v, out_hbm.at[pl.ds(w * 320, 320)])


_LANE4 = None


def _make_sc_score(userel):
    @functools.partial(
        pl.kernel,
        out_type=(jax.ShapeDtypeStruct((2, EP, 16), jnp.float32),
                  jax.ShapeDtypeStruct((2 * NP, 16), jnp.float32)),
        mesh=_MESH,
        scratch_types=[
            pltpu.VMEM((CS,), jnp.int32),
            pltpu.VMEM((CS,), jnp.int32),
            pltpu.VMEM((CS,), jnp.int32),
            pltpu.VMEM((CS // 128, 128), jnp.int32),
            pltpu.VMEM((CS, 16), jnp.float32),
            pltpu.VMEM((CS, 16), jnp.float32),
            pltpu.VMEM((CS, 16), jnp.float32),
            pltpu.VMEM((CS, 16), jnp.float32),
            pltpu.SemaphoreType.DMA,
            pltpu.VMEM_SHARED((NP, 16), jnp.float32),
        ],
        compiler_params=pltpu.CompilerParams(use_tc_tiling_on_sc=False),
    )
    def sc_score(U, V, W16, SRCG, DSTG, DSTP2, RELG, Z16,
                 EX, S, srcb, dstb, relb, dstb2, urows, vrows, wrows,
                 exrows, sem, s_sh):
        c = lax.axis_index("c")
        sid = lax.axis_index("s")
        iota = lax.iota(jnp.int32, 16)
        lane4 = iota < 4
        nrow0 = pl.multiple_of(sid * NW, 128)
        crow0 = pl.multiple_of(c * NP, 128)

        pltpu.sync_copy(Z16.at[pl.ds(nrow0, NW)], s_sh.at[pl.ds(nrow0, NW)])
        plsc.subcore_barrier()

        def chunk(k, _):
            b = pl.multiple_of(sid * EW + k * CS, CS)
            pltpu.sync_copy(SRCG.at[c, pl.ds(b, CS)], srcb)
            pltpu.sync_copy(DSTG.at[c, pl.ds(b, CS)], dstb)
            if userel:
                pltpu.sync_copy(RELG.at[c, pl.ds(b, CS)], relb)
            pltpu.sync_copy(
                DSTP2.at[pl.ds(pl.multiple_of(b // 128, 8), CS // 128)], dstb2)

            ng = CS // 128
            for j in range(ng):
                sl = pl.ds(128 * j, 128)
                pltpu.async_copy(U.at[srcb.at[sl]], urows.at[sl], sem)
                pltpu.async_copy(V.at[dstb.at[sl]], vrows.at[sl], sem)
                if userel:
                    pltpu.async_copy(W16.at[relb.at[sl]], wrows.at[sl], sem)
            for _j in range(ng * (3 if userel else 2)):
                pltpu.make_async_copy(U.at[srcb.at[pl.ds(0, 128)]],
                                      urows.at[pl.ds(0, 128)], sem).wait()

            def edge(j, _):
                p = urows[j, pl.ds(0, 16)] + vrows[j, pl.ds(0, 16)]
                if userel:
                    p = p + wrows[j, pl.ds(0, 16)]
                p = jnp.where(p > 0, p, 0.2 * p)
                e = jnp.exp(p)
                e = jnp.where(lane4, e, 0.0)
                e = jnp.where(b + j < E, e, jnp.zeros((16,), jnp.float32))
                exrows[j, pl.ds(0, 16)] = e
                return _
            lax.fori_loop(0, CS, edge, None)

            pltpu.sync_copy(exrows, EX.at[c, pl.ds(b, CS)])
            for j in range(ng):
                pltpu.sync_copy(exrows.at[pl.ds(128 * j, 128)],
                                s_sh.at[dstb2.at[j]], add=True)
            return _
        lax.fori_loop(0, EW // CS, chunk, None)

        plsc.subcore_barrier()
        pltpu.sync_copy(s_sh.at[pl.ds(nrow0, NW)],
                        S.at[pl.ds(pl.multiple_of(c * NP + sid * NW, 128), NW)])

    return sc_score


_sc_score_rel = _make_sc_score(True)
_sc_score_norel = _make_sc_score(False)


CHB = CH // 128


@functools.partial(
    pl.kernel,
    out_type=tuple(jax.ShapeDtypeStruct((2 * NP, 32), jnp.float32)
                   for _ in range(8)),
    mesh=_MESH,
    scratch_types=[
        pltpu.VMEM((CHB, 128), jnp.int32),
        pltpu.VMEM((CHB, 128), jnp.int32),
        pltpu.VMEM((CHB, 128), jnp.int32),
        pltpu.VMEM((CHB, 128), jnp.int32),
        pltpu.VMEM((CH, 16), jnp.float32),
        pltpu.VMEM((CH, 16), jnp.float32),
        pltpu.VMEM((CH, 32), jnp.float32),
        pltpu.VMEM((CH, 32), jnp.float32),
        pltpu.VMEM((128, 32), jnp.float32),
        pltpu.VMEM((128, 32), jnp.float32),
        pltpu.VMEM((128, 32), jnp.float32),
        pltpu.SemaphoreType.DMA,
        pltpu.SemaphoreType.DMA,
        pltpu.SemaphoreType.DMA,
        pltpu.SemaphoreType.DMA,
        pltpu.SemaphoreType.DMA,
        pltpu.SemaphoreType.DMA,
        pltpu.VMEM_SHARED((NP, 32), jnp.float32),
    ],
    compiler_params=pltpu.CompilerParams(use_tc_tiling_on_sc=False),
)
def _sc_hops(F0, F1, F2, F3, RB0, RB1, RB2, RB3, EX, SRCG3, DSTP2, ZG,
             fA0, fA1, fA2, fA3, fB0, fB1, fB2, fB3,
             srcA, srcB, dstA, dstB, exA, exB, rowsA, rowsB,
             gb, featb, rbb, semIA, semIB, semGA, semGB, semSA, semSB, g_sh):
    c = lax.axis_index("c")
    sid = lax.axis_index("s")
    nrow0 = pl.multiple_of(sid * NW, 128)
    crow0 = pl.multiple_of(c * NP, 128)

    pltpu.sync_copy(ZG.at[pl.ds(nrow0, NW)], g_sh.at[pl.ds(nrow0, NW)])
    plsc.subcore_barrier()

    FEATp = (F0, F1, F2, F3)
    fAq = (fA0, fA1, fA2, fA3)
    fBq = (fB0, fB1, fB2, fB3)
    FS = (FEATp, fAq, fBq, fAq)
    FD = (fAq, fBq, fAq, fBq)
    RBp = (RB0, RB1, RB2, RB3)

    bufs = ((srcA, dstA, exA, rowsA, semIA, semGA, semSA),
            (srcB, dstB, exB, rowsB, semIB, semGB, semSB))

    for hop in range(HOP):
        for p in range(4):
            fsrc = FS[hop][p]
            fdst = FD[hop][p]

            def fire_idx(b, buf):
                src, dst, ex, _rows, semI, _sG, _sS = buf
                r0 = pl.multiple_of(b // 128, CHB)
                return (
                    pltpu.async_copy(SRCG3.at[c, pl.ds(r0, CHB)], src, semI),
                    pltpu.async_copy(DSTP2.at[pl.ds(r0, CHB)], dst, semI),
                    pltpu.async_copy(EX.at[c, pl.ds(b, CH)], ex, semI),
                )

            def fire_gather(buf, fsrc=fsrc):
                src, _d, _e, rows, _sI, semG, _sS = buf
                return tuple(
                    pltpu.async_copy(fsrc.at[src.at[j]],
                                     rows.at[pl.ds(128 * j, 128)], semG)
                    for j in range(CHB))

            def mul_scatter(buf, p=p):
                _s, dst, ex, rows, _sI, _sG, semS = buf

                @plsc.parallel_loop(0, CH, unroll=2)
                def _mul(j):
                    exv = ex[j, pl.ds(0, 16)]
                    sc = exv[p]
                    for q in range(2):
                        sl = pl.ds(q * 16, 16)
                        rows[j, sl] = rows[j, sl] * sc

                return tuple(
                    pltpu.async_copy(rows.at[pl.ds(128 * j, 128)],
                                     g_sh.at[dst.at[j]], semS, add=True)
                    for j in range(CHB))

            def chunk2(k2, _):
                bA = pl.multiple_of(sid * EW + (2 * k2) * CH, CH)
                bB = pl.multiple_of(sid * EW + (2 * k2 + 1) * CH, CH)
                dIA = fire_idx(bA, bufs[0])
                dIB = fire_idx(bB, bufs[1])
                for d in dIA:
                    d.wait()
                dGA = fire_gather(bufs[0])
                for d in dIB:
                    d.wait()
                dGB = fire_gather(bufs[1])
                for d in dGA:
                    d.wait()
                dSA = mul_scatter(bufs[0])
                for d in dGB:
                    d.wait()
                dSB = mul_scatter(bufs[1])
                for d in dSA + dSB:
                    d.wait()
                return _
            lax.fori_loop(0, EW // (2 * CH), chunk2, None)
            plsc.subcore_barrier()

            for q in range(NW // 128):
                r0n = pl.multiple_of(nrow0 + q * 128, 128)
                fr = pl.multiple_of(crow0 + nrow0 + q * 128, 128)
                pltpu.sync_copy(g_sh.at[pl.ds(r0n, 128)], gb)
                pltpu.sync_copy(FEATp[p].at[pl.ds(fr, 128)], featb)
                pltpu.sync_copy(RBp[p].at[pl.ds(fr, 128)], rbb)

                @plsc.parallel_loop(0, 128, unroll=1)
                def _row(i):
                    for m in range(2):
                        sl = pl.ds(m * 16, 16)
                        gb[i, sl] = (ALPHA * featb[i, sl]
                                     + rbb[i, sl] * gb[i, sl])

                pltpu.sync_copy(gb, fdst.at[pl.ds(fr, 128)])
                if not (hop == HOP - 1 and p == 3):
                    pltpu.sync_copy(ZG.at[pl.ds(r0n, 128)],
                                    g_sh.at[pl.ds(r0n, 128)])
            plsc.subcore_barrier()


def _prep_weights(a):
    eye = jnp.eye(H, dtype=jnp.float32)
    def bd(ah):
        full = (eye[:, None, :] * ah[:, :, None]).reshape(H * DH, H)
        halves = jnp.stack([full[:128, :4], full[128:, 4:]])
        return jnp.pad(halves, ((0, 0), (0, 0), (0, 12)))
    return bd(a[0]), bd(a[1])


def kernel(node_table, rel_table, We1, Wr1, a1, Wres1, We2, a2, Wres2,
           ent_ids, rel_ids, edge_index, batch_node_ids):
    src = edge_index[0].astype(jnp.int32)
    dst = edge_index[1].astype(jnp.int32)
    rel = rel_ids.astype(jnp.int32)

    enti = jnp.pad(ent_ids.astype(jnp.int32), (0, NP - N)).reshape(32, 5, 64)
    A0_1, A1_1 = _prep_weights(a1)
    A2_1 = jnp.pad((jnp.eye(H, dtype=jnp.float32)[:, None, :]
                    * a1[2][:, :, None]).reshape(H * DH, H), ((0, 0), (0, 8)))
    A0_2, A1_2 = _prep_weights(a2)
    K1 = jnp.ones((1, 32), jnp.float32)
    ids2d = batch_node_ids.astype(jnp.int32).reshape(64, 1)

    srcp = jnp.pad(src, (0, EP - E))
    dstp = jnp.pad(dst, (0, EP - E))
    relp = jnp.pad(rel, (0, EP - E))
    srcg = jnp.stack([srcp, srcp + NP])
    dstg = jnp.stack([dstp, dstp + NP])
    relg = jnp.stack([relp, relp])
    srcg3 = srcg.reshape(2, EP // 128, 128)
    dstp2 = dstp.reshape(EP // 128, 128)
    z16 = jnp.zeros((NP, 16), jnp.float32)
    zg = jnp.zeros((NP, 32), jnp.float32)
    w16_zero = jnp.zeros((REL, 16), jnp.float32)

    ENT = _sc_embed(node_table, enti)

    w16_1 = _tc_rel(rel_table, Wr1, A2_1)
    w16c_1 = jnp.concatenate(
        [jnp.pad(w16_1[:, :4], ((0, 0), (0, 12))),
         jnp.pad(w16_1[:, 4:8], ((0, 0), (0, 12)))], axis=0)
    FEATq1, U, V, RES1 = _tc_proj(ENT, We1, Wres1, A0_1, A1_1)
    relg1 = jnp.stack([relp, relp + REL])
    EX1, S1 = _sc_score_rel(U, V, w16c_1, srcg, dstg, dstp2, relg1, z16)
    RBq1 = _tc_rb(S1, K1)
    out1 = _sc_hops(*FEATq1, *RBq1, EX1, srcg3, dstp2, zg)
    H1 = _tc_bridge(out1[4:], RES1)

    FEATq2, Ub, Vb, RES2 = _tc_proj(H1, We2, Wres2, A0_2, A1_2)
    EX2, S2 = _sc_score_norel(Ub, Vb, w16_zero, srcg, dstg, dstp2, relg, z16)
    RBq2 = _tc_rb(S2, K1)
    out2 = _sc_hops(*FEATq2, *RBq2, EX2, srcg3, dstp2, zg)

    return _tc_final(out2[4:], RES2, ids2d)
